# baseline (speedup 1.0000x reference)
import functools

import jax
import jax.numpy as jnp
from jax import lax
from jax.experimental import pallas as pl
from jax.experimental.pallas import tpu as pltpu
from jax.experimental.pallas import tpu_sc as plsc

_B, _T = 4096, 25
_ROWS = _B * _T
_NW = 32
_RPW = _ROWS // _NW
_CHUNK = 320
_NCHUNK = _RPW // _CHUNK
_GROUPS = _CHUNK // 16
_BPW = _B // _NW

_OFFSET_VALS = (
    0.0, 132.948591, 442.894612, 454.206447, 162.767078, 74.999437,
    132.948826, 442.894413, 454.20659, 162.767426, 74.999948, 0.1,
    233.383263, 257.077681, 121.134938, 115.002227, 257.077681,
    151.034226, 278.882773, 251.733451, 0.0, 99.999627, 100.000188, 0.0,
    257.077681, 151.031437, 278.892924, 251.72868, 0.0, 99.999888,
    137.499922, 0.0,
)
_DIM_USED = (2, 3, 4, 5, 7, 8, 9, 10, 12, 13, 14, 15, 17, 18, 19, 21, 22,
             25, 26, 27, 29, 30)
_CHAIN_PARENT = {
    2: None, 3: 2, 4: 3, 5: 4,
    7: None, 8: 7, 9: 8, 10: 9,
    12: None, 13: 12, 14: 13, 15: 14,
    17: 13, 18: 17, 19: 18,
    21: 19, 22: 19,
    25: 13, 26: 25, 27: 26,
    29: 27, 30: 27,
}
_COPIES = ((16, 13), (20, 19), (23, 22), (24, 13), (28, 27), (31, 30))
_ZERO_JOINTS = (1, 6, 11)


def _rsqrt(x):
    i = plsc.bitcast(x, jnp.int32)
    i = jnp.int32(0x5F3759DF) - (i >> 1)
    y = plsc.bitcast(i, jnp.float32)
    xh = x * jnp.float32(0.5)
    for _ in range(3):
        y = y * (jnp.float32(1.5) - xh * y * y)
    return y


def _sc_body(obs_hbm, pred_hbm, out_hbm, pred_v, obs_v, out_v):
    cid = lax.axis_index("c")
    sid = lax.axis_index("s")
    wid = sid * 2 + cid
    row0 = wid * _RPW
    b0 = wid * _BPW
    pltpu.sync_copy(obs_hbm.at[pl.ds(b0 * 2, _BPW * 2)], obs_v)
    lane = lax.iota(jnp.int32, 16)

    def splat(v):
        return jnp.full((16,), v, jnp.int32)

    def chunk_body(ci, carry):
        start = row0 + ci * _CHUNK
        pltpu.sync_copy(pred_hbm.at[pl.ds(start * 66, _CHUNK * 66)], pred_v)

        def group_body(g, carry2):
            rows = g * 16 + lane
            bloc = lax.div(start + rows, splat(_T)) - b0
            in_base = rows * 66
            out_base = rows * 96

            def gath(ch):
                return plsc.load_gather(pred_v, [in_base + ch])

            def put(ch, val):
                plsc.store_scatter(out_v, [out_base + ch], val)

            zero = jnp.zeros((16,), jnp.float32)
            obs_base = bloc * 2
            put(0, plsc.load_gather(obs_v, [obs_base]))
            put(1, plsc.load_gather(obs_v, [obs_base + 1]))
            put(2, zero)
            for j in _ZERO_JOINTS:
                put(3 * j, zero)
                put(3 * j + 1, zero)
                put(3 * j + 2, zero)
            saved = {}
            for k, j in enumerate(_DIM_USED):
                x = gath(3 * k)
                y = gath(3 * k + 1)
                z = gath(3 * k + 2)
                s = _rsqrt(x * x + y * y + z * z) * jnp.float32(_OFFSET_VALS[j])
                vx, vy, vz = x * s, y * s, z * s
                p = _CHAIN_PARENT[j]
                if p is not None:
                    px, py, pz = saved[p]
                    vx, vy, vz = px + vx, py + vy, pz + vz
                saved[j] = (vx, vy, vz)
                put(3 * j, vx)
                put(3 * j + 1, vy)
                put(3 * j + 2, vz)
            for t, src in _COPIES:
                sx, sy, sz = saved[src]
                put(3 * t, sx)
                put(3 * t + 1, sy)
                put(3 * t + 2, sz)
            return carry2

        lax.fori_loop(0, _GROUPS, group_body, 0)
        pltpu.sync_copy(out_v, out_hbm.at[pl.ds(start * 96, _CHUNK * 96)])
        return carry

    lax.fori_loop(0, _NCHUNK, chunk_body, 0)


@functools.partial(
    pl.kernel,
    mesh=plsc.VectorSubcoreMesh(core_axis_name="c", subcore_axis_name="s"),
    out_type=jax.ShapeDtypeStruct((_ROWS * 96,), jnp.float32),
    scratch_types=[
        pltpu.VMEM((_CHUNK * 66,), jnp.float32),
        pltpu.VMEM((_BPW * 2,), jnp.float32),
        pltpu.VMEM((_CHUNK * 96,), jnp.float32),
    ],
    compiler_params=pltpu.CompilerParams(needs_layout_passes=False),
)
def _postprocess_sc(obs_hbm, pred_hbm, out_hbm, pred_v, obs_v, out_v):
    _sc_body(obs_hbm, pred_hbm, out_hbm, pred_v, obs_v, out_v)


def kernel(observed_pose, pred_pose):
    obs2 = observed_pose[:, -1, :2].reshape(_B * 2)
    pred2 = pred_pose.reshape(_ROWS * 66)
    out = _postprocess_sc(obs2, pred2)
    return out.reshape(_B, _T, 96)


# speedup vs baseline: 1.2555x; 1.2555x over previous
import functools

import jax
import jax.numpy as jnp
from jax import lax
from jax.experimental import pallas as pl
from jax.experimental.pallas import tpu as pltpu
from jax.experimental.pallas import tpu_sc as plsc

_B, _T = 4096, 25
_ROWS = _B * _T
_NW = 32
_RPW = _ROWS // _NW
_CHUNK = 320
_NCHUNK = _RPW // _CHUNK
_GROUPS = _CHUNK // 16
_BPW = _B // _NW

_OFFSET_VALS = (
    0.0, 132.948591, 442.894612, 454.206447, 162.767078, 74.999437,
    132.948826, 442.894413, 454.20659, 162.767426, 74.999948, 0.1,
    233.383263, 257.077681, 121.134938, 115.002227, 257.077681,
    151.034226, 278.882773, 251.733451, 0.0, 99.999627, 100.000188, 0.0,
    257.077681, 151.031437, 278.892924, 251.72868, 0.0, 99.999888,
    137.499922, 0.0,
)
_DIM_USED = (2, 3, 4, 5, 7, 8, 9, 10, 12, 13, 14, 15, 17, 18, 19, 21, 22,
             25, 26, 27, 29, 30)
_CHAIN_PARENT = {
    2: None, 3: 2, 4: 3, 5: 4,
    7: None, 8: 7, 9: 8, 10: 9,
    12: None, 13: 12, 14: 13, 15: 14,
    17: 13, 18: 17, 19: 18,
    21: 19, 22: 19,
    25: 13, 26: 25, 27: 26,
    29: 27, 30: 27,
}
_COPIES = ((16, 13), (20, 19), (23, 22), (24, 13), (28, 27), (31, 30))
_ZERO_JOINTS = (1, 6, 11)


def _rsqrt(x):
    i = plsc.bitcast(x, jnp.int32)
    i = jnp.int32(0x5F3759DF) - (i >> 1)
    y = plsc.bitcast(i, jnp.float32)
    xh = x * jnp.float32(0.5)
    for _ in range(2):
        y = y * (jnp.float32(1.5) - xh * y * y)
    return y


def _compute_groups(start, pred_v, obs_v, out_v, b0, lane):
    def splat(v):
        return jnp.full((16,), v, jnp.int32)

    @plsc.parallel_loop(0, _GROUPS, unroll=2)
    def group_body(g):
        rows = g * 16 + lane
        bloc = lax.div(start + rows, splat(_T)) - b0
        in_base = rows * 66
        out_base = rows * 96

        def gath(ch):
            return plsc.load_gather(pred_v, [in_base + ch])

        def put(ch, val):
            plsc.store_scatter(out_v, [out_base + ch], val)

        zero = jnp.zeros((16,), jnp.float32)
        obs_base = bloc * 2
        put(0, plsc.load_gather(obs_v, [obs_base]))
        put(1, plsc.load_gather(obs_v, [obs_base + 1]))
        put(2, zero)
        for j in _ZERO_JOINTS:
            put(3 * j, zero)
            put(3 * j + 1, zero)
            put(3 * j + 2, zero)
        saved = {}
        for k, j in enumerate(_DIM_USED):
            x = gath(3 * k)
            y = gath(3 * k + 1)
            z = gath(3 * k + 2)
            s = _rsqrt(x * x + y * y + z * z) * jnp.float32(_OFFSET_VALS[j])
            vx, vy, vz = x * s, y * s, z * s
            p = _CHAIN_PARENT[j]
            if p is not None:
                px, py, pz = saved[p]
                vx, vy, vz = px + vx, py + vy, pz + vz
            saved[j] = (vx, vy, vz)
            put(3 * j, vx)
            put(3 * j + 1, vy)
            put(3 * j + 2, vz)
        for t, src in _COPIES:
            sx, sy, sz = saved[src]
            put(3 * t, sx)
            put(3 * t + 1, sy)
            put(3 * t + 2, sz)


def _sc_body(obs_hbm, pred_hbm, out_hbm, pred_v0, pred_v1, obs_v, out_v0,
             out_v1, sem_i0, sem_i1, sem_o0, sem_o1):
    cid = lax.axis_index("c")
    sid = lax.axis_index("s")
    wid = sid * 2 + cid
    row0 = wid * _RPW
    b0 = wid * _BPW
    pltpu.sync_copy(obs_hbm.at[pl.ds(b0 * 2, _BPW * 2)], obs_v)
    lane = lax.iota(jnp.int32, 16)

    pred_bufs = (pred_v0, pred_v1)
    out_bufs = (out_v0, out_v1)
    sem_ins = (sem_i0, sem_i1)
    sem_outs = (sem_o0, sem_o1)

    def in_copy(ci, par):
        start = row0 + ci * _CHUNK
        return pltpu.make_async_copy(
            pred_hbm.at[pl.ds(start * 66, _CHUNK * 66)], pred_bufs[par],
            sem_ins[par])

    def out_copy(ci, par):
        start = row0 + ci * _CHUNK
        return pltpu.make_async_copy(
            out_bufs[par], out_hbm.at[pl.ds(start * 96, _CHUNK * 96)],
            sem_outs[par])

    in_copy(0, 0).start()
    in_copy(1, 1).start()

    def pair_body(i, carry):
        for par in (0, 1):
            ci = 2 * i + par
            in_copy(ci, par).wait()

            @pl.when(i > 0)
            def _wait_out():
                out_copy(ci - 2, par).wait()

            start = row0 + ci * _CHUNK
            _compute_groups(start, pred_bufs[par], obs_v, out_bufs[par], b0,
                            lane)

            @pl.when(ci + 2 < _NCHUNK)
            def _next_in():
                in_copy(ci + 2, par).start()

            out_copy(ci, par).start()
        return carry

    lax.fori_loop(0, _NCHUNK // 2, pair_body, 0)
    out_copy(_NCHUNK - 2, 0).wait()
    out_copy(_NCHUNK - 1, 1).wait()


@functools.partial(
    pl.kernel,
    mesh=plsc.VectorSubcoreMesh(core_axis_name="c", subcore_axis_name="s"),
    out_type=jax.ShapeDtypeStruct((_ROWS * 96,), jnp.float32),
    scratch_types=[
        pltpu.VMEM((_CHUNK * 66,), jnp.float32),
        pltpu.VMEM((_CHUNK * 66,), jnp.float32),
        pltpu.VMEM((_BPW * 2,), jnp.float32),
        pltpu.VMEM((_CHUNK * 96,), jnp.float32),
        pltpu.VMEM((_CHUNK * 96,), jnp.float32),
        pltpu.SemaphoreType.DMA,
        pltpu.SemaphoreType.DMA,
        pltpu.SemaphoreType.DMA,
        pltpu.SemaphoreType.DMA,
    ],
    compiler_params=pltpu.CompilerParams(needs_layout_passes=False),
)
def _postprocess_sc(obs_hbm, pred_hbm, out_hbm, pred_v0, pred_v1, obs_v,
                    out_v0, out_v1, sem_i0, sem_i1, sem_o0, sem_o1):
    _sc_body(obs_hbm, pred_hbm, out_hbm, pred_v0, pred_v1, obs_v, out_v0,
             out_v1, sem_i0, sem_i1, sem_o0, sem_o1)


def kernel(observed_pose, pred_pose):
    obs2 = observed_pose[:, -1, :2].reshape(_B * 2)
    pred2 = pred_pose.reshape(_ROWS * 66)
    out = _postprocess_sc(obs2, pred2)
    return out.reshape(_B, _T, 96)
